# baseline (speedup 1.0000x reference)
import functools

import jax
import jax.numpy as jnp
from jax import lax
from jax.experimental import pallas as pl
from jax.experimental.pallas import tpu as pltpu
from jax.experimental.pallas import tpu_sc as plsc

QA = 128
QB = 256
CT = 16384
G = 128
K = 20
T = 22
TP = 24
NEG = -1e38
IMAX = 2**31 - 1
BIGF = 1e9

NC = 2
NS = 16
NW = NC * NS


def _phase_a(x_ref, c_ref, d_ref, gm_ref):
    q = x_ref[...]
    c = c_ref[...]
    p = jax.lax.dot_general(q, c, (((1,), (1,)), ((), ())),
                            preferred_element_type=jnp.float32)
    inner = -2.0 * p
    qq = jnp.sum(q * q, axis=1, keepdims=True)
    csq = c * c
    ones8 = jnp.ones((8, c.shape[1]), jnp.float32)
    cc8 = jax.lax.dot_general(ones8, csq, (((1,), (1,)), ((), ())),
                              precision=jax.lax.Precision.HIGHEST,
                              preferred_element_type=jnp.float32)
    cc = cc8[0:1, :]
    d = -((cc + inner) + qq)
    d3 = d.reshape(QA, CT // G, G)
    d_ref[...] = d3
    gm_ref[...] = jnp.max(d3, axis=2)


def _phase_b(gm_ref, gid_ref, *, ng):
    w = gm_ref[...]
    li = jax.lax.broadcasted_iota(jnp.int32, (QB, ng), 1)
    lane = jax.lax.broadcasted_iota(jnp.int32, (QB, TP), 1)
    gacc = jnp.full((QB, TP), ng - 1, jnp.int32)
    for t in range(T):
        m = jnp.max(w, axis=1, keepdims=True)
        sel = jnp.where(w == m, li, IMAX)
        gmin = jnp.min(sel, axis=1, keepdims=True)
        gacc = jnp.where(lane == t, gmin, gacc)
        w = jnp.where((w == m) & (li == gmin), NEG, w)
    gid_ref[...] = gacc


def _phase_d(vm_ref, gid_ref, oi_ref, ov_ref):
    W = TP * G
    vals = vm_ref[...].reshape(QB, W)
    gidf = gid_ref[...].astype(jnp.float32)
    gexp = jnp.broadcast_to(gidf[:, :, None], (QB, TP, G)).reshape(QB, W)
    lmod = (jax.lax.broadcasted_iota(jnp.int32, (QB, W), 1)
            & (G - 1)).astype(jnp.float32)
    idxm = gexp * float(G) + lmod
    lane = jax.lax.broadcasted_iota(jnp.int32, (QB, 128), 1)
    new_v = jnp.full((QB, 128), NEG, jnp.float32)
    new_i = jnp.full((QB, 128), BIGF, jnp.float32)
    for t in range(K):
        m = jnp.max(vals, axis=1, keepdims=True)
        sel = jnp.where(vals == m, idxm, BIGF)
        amin = jnp.min(sel, axis=1, keepdims=True)
        new_v = jnp.where(lane == t, m, new_v)
        new_i = jnp.where(lane == t, amin, new_i)
        vals = jnp.where((vals == m) & (idxm == amin), NEG, vals)
    ov_ref[...] = new_v
    oi_ref[...] = new_i


def _tc_phases_ab(x, x_neig):
    Q, F = x.shape
    N = x_neig.shape[0]
    n_tiles = (N + CT - 1) // CT
    n_pad = n_tiles * CT
    if n_pad != N:
        x_neig = jnp.pad(x_neig, ((0, n_pad - N), (0, 0)),
                         constant_values=1e18)
    ng = n_pad // G

    d3, gm = pl.pallas_call(
        _phase_a,
        grid=(Q // QA, n_tiles),
        in_specs=[
            pl.BlockSpec((QA, F), lambda i, j: (i, 0)),
            pl.BlockSpec((CT, F), lambda i, j: (j, 0)),
        ],
        out_specs=[
            pl.BlockSpec((QA, CT // G, G), lambda i, j: (i, j, 0)),
            pl.BlockSpec((QA, CT // G), lambda i, j: (i, j)),
        ],
        out_shape=[
            jax.ShapeDtypeStruct((Q, ng, G), jnp.float32),
            jax.ShapeDtypeStruct((Q, ng), jnp.float32),
        ],
    )(x, x_neig)

    gids = pl.pallas_call(
        functools.partial(_phase_b, ng=ng),
        grid=(Q // QB,),
        in_specs=[pl.BlockSpec((QB, ng), lambda i: (i, 0))],
        out_specs=pl.BlockSpec((QB, TP), lambda i: (i, 0)),
        out_shape=jax.ShapeDtypeStruct((Q, TP), jnp.int32),
    )(gm)
    return d3, gids, ng


def _make_phase_c(Q, ng):
    QPW = Q // NW
    mesh = plsc.VectorSubcoreMesh(core_axis_name="c", subcore_axis_name="s")

    @functools.partial(
        pl.kernel, mesh=mesh,
        out_type=jax.ShapeDtypeStruct((Q, TP, G), jnp.float32),
        scratch_types=[
            pltpu.VMEM((TP,), jnp.int32),
            pltpu.VMEM((TP,), jnp.int32),
            pltpu.VMEM((TP, G), jnp.float32),
            pltpu.SemaphoreType.DMA,
        ],
    )
    def phase_c(dg_hbm, gids_hbm, outv_hbm, gid_v, rid_v, rows_v, sem):
        wid = lax.axis_index("s") * NC + lax.axis_index("c")
        base = wid * QPW

        def body(qi, carry):
            q = base + qi
            pltpu.sync_copy(gids_hbm.at[q], gid_v)
            qoff = q * ng
            rid_v[pl.ds(0, 16)] = gid_v[pl.ds(0, 16)] + qoff
            rid_v[pl.ds(8, 16)] = gid_v[pl.ds(8, 16)] + qoff
            pltpu.make_async_copy(dg_hbm.at[rid_v], rows_v, sem).start()
            pltpu.make_async_copy(dg_hbm.at[rid_v], rows_v, sem).wait()
            pltpu.sync_copy(rows_v, outv_hbm.at[q])
            return carry

        lax.fori_loop(0, QPW, body, jnp.int32(0))

    return phase_c


def kernel(x, x_neig, k):
    del k
    Q = x.shape[0]
    d3, gids, ng = _tc_phases_ab(x, x_neig)

    dg = d3.reshape(Q * ng, G)
    vm = _make_phase_c(Q, ng)(dg, gids)

    out_i, out_v = pl.pallas_call(
        _phase_d,
        grid=(Q // QB,),
        in_specs=[
            pl.BlockSpec((QB, TP, G), lambda i: (i, 0, 0)),
            pl.BlockSpec((QB, TP), lambda i: (i, 0)),
        ],
        out_specs=[
            pl.BlockSpec((QB, 128), lambda i: (i, 0)),
            pl.BlockSpec((QB, 128), lambda i: (i, 0)),
        ],
        out_shape=[
            jax.ShapeDtypeStruct((Q, 128), jnp.float32),
            jax.ShapeDtypeStruct((Q, 128), jnp.float32),
        ],
    )(vm, gids)
    return (out_i[:, :K], out_v[:, :K])


# speedup vs baseline: 1.0142x; 1.0142x over previous
import functools

import jax
import jax.numpy as jnp
from jax import lax
from jax.experimental import pallas as pl
from jax.experimental.pallas import tpu as pltpu
from jax.experimental.pallas import tpu_sc as plsc

QA = 128
QB = 256
CT = 16384
G = 128
K = 20
T = 22
TP = 24
NEG = -1e38
IMAX = 2**31 - 1
BIGF = 1e9

NC = 2
NS = 16
NW = NC * NS


def _phase_a(x_ref, c_ref, d_ref, gm_ref, *, n_valid):
    j = pl.program_id(1)
    q = x_ref[...]
    c = c_ref[...]
    p = jax.lax.dot_general(q, c, (((1,), (1,)), ((), ())),
                            preferred_element_type=jnp.float32)
    inner = -2.0 * p
    qq = jnp.sum(q * q, axis=1, keepdims=True)
    csq = c * c
    ones8 = jnp.ones((8, c.shape[1]), jnp.float32)
    cc8 = jax.lax.dot_general(ones8, csq, (((1,), (1,)), ((), ())),
                              precision=jax.lax.Precision.HIGHEST,
                              preferred_element_type=jnp.float32)
    cc = cc8[0:1, :]
    d = -((cc + inner) + qq)
    gidx = j * CT + jax.lax.broadcasted_iota(jnp.int32, (QA, CT), 1)
    d = jnp.where(gidx < n_valid, d, NEG)
    d3 = d.reshape(QA, CT // G, G)
    d_ref[...] = d3
    gm_ref[...] = jnp.max(d3, axis=2)


def _phase_b(gm_ref, gid_ref, *, ng):
    w = gm_ref[...]
    li = jax.lax.broadcasted_iota(jnp.int32, (QB, ng), 1)
    lane = jax.lax.broadcasted_iota(jnp.int32, (QB, TP), 1)
    gacc = jnp.full((QB, TP), ng - 1, jnp.int32)
    for t in range(T):
        m = jnp.max(w, axis=1, keepdims=True)
        sel = jnp.where(w == m, li, IMAX)
        gmin = jnp.min(sel, axis=1, keepdims=True)
        gacc = jnp.where(lane == t, gmin, gacc)
        w = jnp.where((w == m) & (li == gmin), NEG, w)
    gid_ref[...] = gacc


def _phase_d(vm_ref, gid_ref, oi_ref, ov_ref):
    W = TP * G
    vals = vm_ref[...].reshape(QB, W)
    gidf = gid_ref[...].astype(jnp.float32)
    gexp = jnp.broadcast_to(gidf[:, :, None], (QB, TP, G)).reshape(QB, W)
    lmod = (jax.lax.broadcasted_iota(jnp.int32, (QB, W), 1)
            & (G - 1)).astype(jnp.float32)
    idxm = gexp * float(G) + lmod
    lane = jax.lax.broadcasted_iota(jnp.int32, (QB, 128), 1)
    new_v = jnp.full((QB, 128), NEG, jnp.float32)
    new_i = jnp.full((QB, 128), BIGF, jnp.float32)
    for t in range(K):
        m = jnp.max(vals, axis=1, keepdims=True)
        sel = jnp.where(vals == m, idxm, BIGF)
        amin = jnp.min(sel, axis=1, keepdims=True)
        new_v = jnp.where(lane == t, m, new_v)
        new_i = jnp.where(lane == t, amin, new_i)
        vals = jnp.where((vals == m) & (idxm == amin), NEG, vals)
    ov_ref[...] = new_v
    oi_ref[...] = new_i


def _tc_phases_ab(x, x_neig):
    Q, F = x.shape
    N = x_neig.shape[0]
    n_tiles = (N + CT - 1) // CT
    n_pad = n_tiles * CT
    if n_pad != N:
        x_neig = jnp.pad(x_neig, ((0, n_pad - N), (0, 0)))
    ng = n_pad // G

    d3, gm = pl.pallas_call(
        functools.partial(_phase_a, n_valid=N),
        grid=(Q // QA, n_tiles),
        in_specs=[
            pl.BlockSpec((QA, F), lambda i, j: (i, 0)),
            pl.BlockSpec((CT, F), lambda i, j: (j, 0)),
        ],
        out_specs=[
            pl.BlockSpec((QA, CT // G, G), lambda i, j: (i, j, 0)),
            pl.BlockSpec((QA, CT // G), lambda i, j: (i, j)),
        ],
        out_shape=[
            jax.ShapeDtypeStruct((Q, ng, G), jnp.float32),
            jax.ShapeDtypeStruct((Q, ng), jnp.float32),
        ],
    )(x, x_neig)

    gids = pl.pallas_call(
        functools.partial(_phase_b, ng=ng),
        grid=(Q // QB,),
        in_specs=[pl.BlockSpec((QB, ng), lambda i: (i, 0))],
        out_specs=pl.BlockSpec((QB, TP), lambda i: (i, 0)),
        out_shape=jax.ShapeDtypeStruct((Q, TP), jnp.int32),
    )(gm)
    return d3, gids, ng


def _make_phase_c(Q, ng):
    QPW = Q // NW
    mesh = plsc.VectorSubcoreMesh(core_axis_name="c", subcore_axis_name="s")

    @functools.partial(
        pl.kernel, mesh=mesh,
        out_type=jax.ShapeDtypeStruct((Q, TP, G), jnp.float32),
        scratch_types=[
            pltpu.VMEM((TP,), jnp.int32),
            pltpu.VMEM((TP,), jnp.int32),
            pltpu.VMEM((TP, G), jnp.float32),
            pltpu.SemaphoreType.DMA,
        ],
    )
    def phase_c(dg_hbm, gids_hbm, outv_hbm, gid_v, rid_v, rows_v, sem):
        wid = lax.axis_index("s") * NC + lax.axis_index("c")
        base = wid * QPW

        def body(qi, carry):
            q = base + qi
            pltpu.sync_copy(gids_hbm.at[q], gid_v)
            qoff = q * ng
            rid_v[pl.ds(0, 16)] = gid_v[pl.ds(0, 16)] + qoff
            rid_v[pl.ds(8, 16)] = gid_v[pl.ds(8, 16)] + qoff
            pltpu.make_async_copy(dg_hbm.at[rid_v], rows_v, sem).start()
            pltpu.make_async_copy(dg_hbm.at[rid_v], rows_v, sem).wait()
            pltpu.sync_copy(rows_v, outv_hbm.at[q])
            return carry

        lax.fori_loop(0, QPW, body, jnp.int32(0))

    return phase_c


def kernel(x, x_neig, k):
    del k
    Q = x.shape[0]
    d3, gids, ng = _tc_phases_ab(x, x_neig)

    dg = d3.reshape(Q * ng, G)
    vm = _make_phase_c(Q, ng)(dg, gids)

    out_i, out_v = pl.pallas_call(
        _phase_d,
        grid=(Q // QB,),
        in_specs=[
            pl.BlockSpec((QB, TP, G), lambda i: (i, 0, 0)),
            pl.BlockSpec((QB, TP), lambda i: (i, 0)),
        ],
        out_specs=[
            pl.BlockSpec((QB, 128), lambda i: (i, 0)),
            pl.BlockSpec((QB, 128), lambda i: (i, 0)),
        ],
        out_shape=[
            jax.ShapeDtypeStruct((Q, 128), jnp.float32),
            jax.ShapeDtypeStruct((Q, 128), jnp.float32),
        ],
    )(vm, gids)
    return (out_i[:, :K], out_v[:, :K])
